# baseline (speedup 1.0000x reference)
import functools

import jax
import jax.numpy as jnp
from jax import lax
from jax.experimental import pallas as pl
from jax.experimental.pallas import tpu as pltpu
from jax.experimental.pallas import tpu_sc as plsc

BATCH = 4096
HIST = 50
DIM = 128
TOTAL = BATCH * HIST
CATE_VOCAB = 1000
CATE_PAD = 1024

_info = plsc.get_sparse_core_info()
_NC, _NS = _info.num_cores, _info.num_subcores
_NW = _NC * _NS
_PER_W = TOTAL // _NW
_CHUNK = 128
_NCHUNK = _PER_W // _CHUNK
_NBUF = 5
_NGROUP = _NCHUNK // _NBUF

_mesh = plsc.VectorSubcoreMesh(core_axis_name="c", subcore_axis_name="s")


@functools.partial(
    pl.kernel,
    mesh=_mesh,
    out_type=jax.ShapeDtypeStruct((TOTAL, DIM), jnp.float32),
    scratch_types=(
        [pltpu.VMEM((_PER_W,), jnp.int32)]
        + [pltpu.VMEM((_CHUNK, DIM), jnp.float32) for _ in range(_NBUF)]
        + [pltpu.SemaphoreType.DMA for _ in range(2 * _NBUF)]
    ),
)
def _item_sc_kernel(idx_hbm, tab_hbm, out_hbm, idx_v, *bufs_and_sems):
    rows = bufs_and_sems[:_NBUF]
    gsem = bufs_and_sems[_NBUF:2 * _NBUF]
    ssem = bufs_and_sems[2 * _NBUF:]
    wid = lax.axis_index("s") * _NC + lax.axis_index("c")
    base = wid * _PER_W

    pltpu.sync_copy(idx_hbm.at[pl.ds(base, _PER_W)], idx_v)

    def gather_desc(i, b):
        off = pl.multiple_of(i * _CHUNK, _CHUNK)
        return pltpu.make_async_copy(
            tab_hbm.at[idx_v.at[pl.ds(off, _CHUNK)]], rows[b], gsem[b])

    def store_desc(i, b):
        off = pl.multiple_of(base + i * _CHUNK, _CHUNK)
        return pltpu.make_async_copy(
            rows[b], out_hbm.at[pl.ds(off, _CHUNK)], ssem[b])

    for b in range(_NBUF - 1):
        gather_desc(b, b).start()

    def body(q, carry):
        for b in range(_NBUF):
            i = q * _NBUF + b
            gather_desc(i, b).wait()
            store_desc(i, b).start()
            nb = (b + _NBUF - 1) % _NBUF
            if b == 0:
                @pl.when(q > 0)
                def _():
                    store_desc(i - 1, nb).wait()
                    gather_desc(i + _NBUF - 1, nb).start()
                @pl.when(q == 0)
                def _():
                    gather_desc(i + _NBUF - 1, nb).start()
            else:
                store_desc(i - 1, nb).wait()
                @pl.when(i + _NBUF - 1 < _NCHUNK)
                def _():
                    gather_desc(i + _NBUF - 1, nb).start()
        return carry

    lax.fori_loop(0, _NGROUP, body, 0)
    store_desc(_NCHUNK - 1, (_NCHUNK - 1) % _NBUF).wait()


_TC_ROWS = 512
_TC_GRID = TOTAL // _TC_ROWS


def _cate_tc_body(idx_ref, tab_ref, out_ref):
    ids = idx_ref[0, 0]
    cols = lax.broadcasted_iota(jnp.int32, (_TC_ROWS, CATE_PAD), 1)
    onehot = (ids[:, None] == cols).astype(jnp.bfloat16)
    out_ref[...] = jnp.dot(onehot, tab_ref[...],
                           preferred_element_type=jnp.float32)


_cate_tc_kernel = pl.pallas_call(
    _cate_tc_body,
    grid=(_TC_GRID,),
    in_specs=[
        pl.BlockSpec((1, 1, _TC_ROWS), lambda i: (i, 0, 0)),
        pl.BlockSpec((CATE_PAD, DIM), lambda i: (0, 0)),
    ],
    out_specs=pl.BlockSpec((_TC_ROWS, DIM), lambda i: (i, 0)),
    out_shape=jax.ShapeDtypeStruct((TOTAL, DIM), jnp.float32),
)


def kernel(indices_item, indices_cate, item_table, cate_table):
    item_vals = _item_sc_kernel(indices_item.reshape(-1), item_table)
    tab_pad = jnp.zeros((CATE_PAD, DIM), jnp.bfloat16).at[:CATE_VOCAB].set(
        cate_table.astype(jnp.bfloat16))
    cate_vals = _cate_tc_kernel(
        indices_cate.reshape(_TC_GRID, 1, _TC_ROWS), tab_pad)
    return item_vals, cate_vals


# speedup vs baseline: 2.2557x; 2.2557x over previous
import functools

import jax
import jax.numpy as jnp
from jax import lax
from jax.experimental import pallas as pl
from jax.experimental.pallas import tpu as pltpu
from jax.experimental.pallas import tpu_sc as plsc

BATCH = 4096
HIST = 50
DIM = 128
TOTAL = BATCH * HIST
CATE_VOCAB = 1000

_info = plsc.get_sparse_core_info()
_NC, _NS = _info.num_cores, _info.num_subcores
_NW = _NC * _NS
_PER_W = TOTAL // _NW
_CHUNK = 64
_NCHUNK = _PER_W // _CHUNK
_NBUF = 10
_NGROUP = _NCHUNK // _NBUF

_mesh = plsc.VectorSubcoreMesh(core_axis_name="c", subcore_axis_name="s")


@functools.partial(
    pl.kernel,
    mesh=_mesh,
    out_type=(
        jax.ShapeDtypeStruct((TOTAL, DIM), jnp.float32),
        jax.ShapeDtypeStruct((TOTAL, DIM), jnp.float32),
    ),
    scratch_types=(
        [pltpu.VMEM((_PER_W,), jnp.int32)]
        + [pltpu.VMEM((_CHUNK, DIM), jnp.float32) for _ in range(_NBUF)]
        + [pltpu.VMEM_SHARED((CATE_VOCAB, DIM), jnp.float32)]
        + [pltpu.SemaphoreType.DMA for _ in range(2 * _NBUF + 1)]
    ),
)
def _gather_kernel(idx_item_hbm, idx_cate_hbm, item_tab_hbm, cate_tab_hbm,
                   out_item_hbm, out_cate_hbm, idx_v, *bufs_and_sems):
    rows = bufs_and_sems[:_NBUF]
    cate_spmem = bufs_and_sems[_NBUF]
    gsem = bufs_and_sems[_NBUF + 1:2 * _NBUF + 1]
    ssem = bufs_and_sems[2 * _NBUF + 1:3 * _NBUF + 1]
    stsem = bufs_and_sems[3 * _NBUF + 1]
    sid = lax.axis_index("s")
    wid = sid * _NC + lax.axis_index("c")
    base = wid * _PER_W

    def do_table(idx_hbm, tab, out_hbm):
        pltpu.sync_copy(idx_hbm.at[pl.ds(base, _PER_W)], idx_v)

        def gather_desc(i, b):
            off = pl.multiple_of(i * _CHUNK, _CHUNK)
            return pltpu.make_async_copy(
                tab.at[idx_v.at[pl.ds(off, _CHUNK)]], rows[b], gsem[b])

        def store_desc(i, b):
            off = pl.multiple_of(base + i * _CHUNK, _CHUNK)
            return pltpu.make_async_copy(
                rows[b], out_hbm.at[pl.ds(off, _CHUNK)], ssem[b])

        for b in range(_NBUF - 1):
            gather_desc(b, b).start()

        def body(q, carry):
            for b in range(_NBUF):
                i = q * _NBUF + b
                gather_desc(i, b).wait()
                store_desc(i, b).start()
                nb = (b + _NBUF - 1) % _NBUF
                if b == 0:
                    @pl.when(q > 0)
                    def _():
                        store_desc(i - 1, nb).wait()
                        gather_desc(i + _NBUF - 1, nb).start()
                    @pl.when(q == 0)
                    def _():
                        gather_desc(i + _NBUF - 1, nb).start()
                else:
                    store_desc(i - 1, nb).wait()
                    @pl.when(i + _NBUF - 1 < _NCHUNK)
                    def _():
                        gather_desc(i + _NBUF - 1, nb).start()
            return carry

        lax.fori_loop(0, _NGROUP, body, 0)
        store_desc(_NCHUNK - 1, (_NCHUNK - 1) % _NBUF).wait()

    stage = pltpu.make_async_copy(cate_tab_hbm, cate_spmem, stsem)

    @pl.when(sid == 0)
    def _():
        stage.start()

    do_table(idx_item_hbm, item_tab_hbm, out_item_hbm)

    @pl.when(sid == 0)
    def _():
        stage.wait()

    plsc.subcore_barrier()
    do_table(idx_cate_hbm, cate_spmem, out_cate_hbm)


def kernel(indices_item, indices_cate, item_table, cate_table):
    item_vals, cate_vals = _gather_kernel(
        indices_item.reshape(-1), indices_cate.reshape(-1),
        item_table, cate_table)
    return item_vals, cate_vals
